# Initial kernel scaffold; baseline (speedup 1.0000x reference)
import jax
import jax.numpy as jnp
from jax.experimental import pallas as pl


def kernel(x, edge_index, W_in, b_in, W_out, b_out, W_root, b_root):
    raise NotImplementedError("write your pallas kernel here")


# speedup vs baseline: 4.0456x; 4.0456x over previous
import functools

import jax
import jax.numpy as jnp
from jax import lax
from jax.experimental import pallas as pl
from jax.experimental.pallas import tpu as pltpu
from jax.experimental.pallas import tpu_sc as plsc

N_NODES = 10000
D = 256
HALF = 128
ROWW = 144
NC = 2
NS = 16
B = 128
N_ACC = 10240
CHUNK = N_ACC // NS


def _sc_segment_sums(xcat, garr, tarr, zer, n_batches):
    mesh = plsc.VectorSubcoreMesh(core_axis_name="c", subcore_axis_name="s")

    @functools.partial(
        pl.kernel,
        out_type=jax.ShapeDtypeStruct((2, NC, N_ACC, ROWW), jnp.float32),
        mesh=mesh,
        scratch_types=[
            pltpu.VMEM_SHARED((N_ACC, ROWW), jnp.float32),
            pltpu.VMEM((n_batches, B), jnp.int32),
            pltpu.VMEM((n_batches, B), jnp.int32),
            pltpu.VMEM((B, ROWW), jnp.float32),
            pltpu.SemaphoreType.DMA,
        ],
        compiler_params=pltpu.CompilerParams(use_tc_tiling_on_sc=False),
    )
    def k(xcat_hbm, garr_hbm, tarr_hbm, zer_hbm, out_hbm,
          acc_sh, gidx_v, tgt_v, rows_v, sem):
        c = lax.axis_index("c")
        s = lax.axis_index("s")
        row0 = pl.multiple_of(s * CHUNK, 8)
        pltpu.sync_copy(zer_hbm.at[pl.ds(0, CHUNK)],
                        acc_sh.at[pl.ds(row0, CHUNK)])
        plsc.subcore_barrier()
        for d in range(2):
            pltpu.sync_copy(garr_hbm.at[d, c, s], gidx_v)
            pltpu.sync_copy(tarr_hbm.at[d, s], tgt_v)

            def step(i, carry):
                pltpu.async_copy(xcat_hbm.at[gidx_v.at[i]], rows_v, sem).wait()
                pltpu.sync_copy(rows_v, acc_sh.at[tgt_v.at[i]], add=True)
                return carry

            lax.fori_loop(0, n_batches, step, 0)
            plsc.subcore_barrier()
            pltpu.sync_copy(acc_sh.at[pl.ds(row0, CHUNK)],
                            out_hbm.at[d, c, pl.ds(row0, CHUNK)])
            if d == 0:
                pltpu.sync_copy(zer_hbm.at[pl.ds(0, CHUNK)],
                                acc_sh.at[pl.ds(row0, CHUNK)])
                plsc.subcore_barrier()

    return k(xcat, garr, tarr, zer)


def _tc_combine(agg, x, W_in, b_in, W_out, b_out, W_root, b_root):
    blk = 400
    grid = (N_NODES // blk,)

    def body(agg_ref, x_ref, wi_ref, wo_ref, wr_ref, b_ref, o_ref):
        dot = functools.partial(
            lax.dot_general,
            dimension_numbers=(((1,), (0,)), ((), ())),
            precision=lax.Precision.HIGHEST,
            preferred_element_type=jnp.float32,
        )
        acc = dot(x_ref[...], wr_ref[...])
        for d, w_ref in ((0, wi_ref), (1, wo_ref)):
            cnt = agg_ref[d, 0, :, HALF:HALF + 1]
            scale = 0.5 / jnp.maximum(cnt, 1.0)
            for c in range(2):
                h = agg_ref[d, c, :, 0:HALF] * scale
                acc = acc + dot(h, w_ref[c * HALF:(c + 1) * HALF, :])
        o_ref[...] = acc + b_ref[...]

    bias = (0.5 * b_in + 0.5 * b_out + b_root).reshape(1, D)
    return pl.pallas_call(
        body,
        grid=grid,
        in_specs=[
            pl.BlockSpec((2, NC, blk, ROWW), lambda i: (0, 0, i, 0)),

            pl.BlockSpec((blk, D), lambda i: (i, 0)),
            pl.BlockSpec((D, D), lambda i: (0, 0)),
            pl.BlockSpec((D, D), lambda i: (0, 0)),
            pl.BlockSpec((D, D), lambda i: (0, 0)),
            pl.BlockSpec((1, D), lambda i: (0, 0)),
        ],
        out_specs=pl.BlockSpec((blk, D), lambda i: (i, 0)),
        out_shape=jax.ShapeDtypeStruct((N_NODES, D), jnp.float32),
    )(agg, x, W_in, W_out, W_root, bias)


def kernel(x, edge_index, W_in, b_in, W_out, b_out, W_root, b_root):
    n_edges = edge_index.shape[1]
    src = edge_index[0].astype(jnp.int32)
    dst = edge_index[1].astype(jnp.int32)

    ones = jnp.ones((N_NODES, 1), jnp.float32)
    zpad = jnp.zeros((N_NODES, ROWW - HALF - 1), jnp.float32)
    xcat = jnp.concatenate([
        jnp.concatenate([x[:, :HALF], ones, zpad], axis=1),
        jnp.concatenate([x[:, HALF:], ones, zpad], axis=1),
        jnp.zeros((8, ROWW), jnp.float32),
    ], axis=0)

    ept = ((n_edges // NS) + B - 1) // B * B
    n_batches = ept // B
    e_pad = NS * ept
    pad = e_pad - n_edges
    pad_g = 2 * N_NODES + (jnp.arange(pad, dtype=jnp.int32) % 8)
    pad_t = jnp.arange(pad, dtype=jnp.int32) % N_NODES

    def tiled(idx, pad_vals):
        return jnp.concatenate([idx, pad_vals]).reshape(NS, n_batches, B)

    garr = jnp.stack([
        jnp.stack([tiled(src, pad_g), tiled(src + N_NODES, pad_g)]),
        jnp.stack([tiled(dst, pad_g), tiled(dst + N_NODES, pad_g)]),
    ])
    tarr = jnp.stack([tiled(dst, pad_t), tiled(src, pad_t)])

    zer = jnp.zeros((CHUNK, ROWW), jnp.float32)
    agg = _sc_segment_sums(xcat, garr, tarr, zer, n_batches)
    return _tc_combine(agg, x, W_in, b_in, W_out, b_out, W_root, b_root)


# speedup vs baseline: 4.2321x; 1.0461x over previous
import functools

import jax
import jax.numpy as jnp
from jax import lax
from jax.experimental import pallas as pl
from jax.experimental.pallas import tpu as pltpu
from jax.experimental.pallas import tpu_sc as plsc

N_NODES = 10000
D = 256
HALF = 128
ROWW = 144
NC = 2
NS = 16
B = 64
N_ACC = 10240
CHUNK = N_ACC // NS


def _sc_segment_sums(xcat, garr, tarr, zer, n_batches):
    mesh = plsc.VectorSubcoreMesh(core_axis_name="c", subcore_axis_name="s")

    @functools.partial(
        pl.kernel,
        out_type=jax.ShapeDtypeStruct((2, NC, N_ACC, ROWW), jnp.float32),
        mesh=mesh,
        scratch_types=[
            pltpu.VMEM_SHARED((N_ACC, ROWW), jnp.float32),
            pltpu.VMEM((n_batches, B), jnp.int32),
            pltpu.VMEM((n_batches, B), jnp.int32),
            pltpu.VMEM((B, ROWW), jnp.float32),
            pltpu.VMEM((B, ROWW), jnp.float32),
            pltpu.SemaphoreType.DMA,
            pltpu.SemaphoreType.DMA,
            pltpu.SemaphoreType.DMA,
            pltpu.SemaphoreType.DMA,
        ],
        compiler_params=pltpu.CompilerParams(use_tc_tiling_on_sc=False),
    )
    def k(xcat_hbm, garr_hbm, tarr_hbm, zer_hbm, out_hbm,
          acc_sh, gidx_v, tgt_v, rows0, rows1, sg0, sg1, ss0, ss1):
        c = lax.axis_index("c")
        s = lax.axis_index("s")
        row0 = pl.multiple_of(s * CHUNK, 8)
        pltpu.sync_copy(zer_hbm.at[pl.ds(0, CHUNK)],
                        acc_sh.at[pl.ds(row0, CHUNK)])
        plsc.subcore_barrier()
        for d in range(2):
            pltpu.sync_copy(garr_hbm.at[d, c, s], gidx_v)
            pltpu.sync_copy(tarr_hbm.at[d, s], tgt_v)

            pltpu.async_copy(xcat_hbm.at[gidx_v.at[0]], rows0, sg0)
            pltpu.async_copy(xcat_hbm.at[gidx_v.at[1]], rows1, sg1)

            def pair(i, carry):
                b = 2 * i
                pltpu.make_async_copy(xcat_hbm.at[gidx_v.at[b]], rows0, sg0).wait()
                cs0 = pltpu.async_copy(rows0, acc_sh.at[tgt_v.at[b]], ss0, add=True)
                pltpu.make_async_copy(xcat_hbm.at[gidx_v.at[b + 1]], rows1, sg1).wait()
                cs1 = pltpu.async_copy(rows1, acc_sh.at[tgt_v.at[b + 1]], ss1, add=True)
                cs0.wait()

                @pl.when(b + 2 < n_batches)
                def _():
                    pltpu.async_copy(xcat_hbm.at[gidx_v.at[b + 2]], rows0, sg0)
                cs1.wait()

                @pl.when(b + 3 < n_batches)
                def _():
                    pltpu.async_copy(xcat_hbm.at[gidx_v.at[b + 3]], rows1, sg1)
                return carry

            lax.fori_loop(0, n_batches // 2, pair, 0)
            plsc.subcore_barrier()
            pltpu.sync_copy(acc_sh.at[pl.ds(row0, CHUNK)],
                            out_hbm.at[d, c, pl.ds(row0, CHUNK)])
            if d == 0:
                pltpu.sync_copy(zer_hbm.at[pl.ds(0, CHUNK)],
                                acc_sh.at[pl.ds(row0, CHUNK)])
                plsc.subcore_barrier()

    return k(xcat, garr, tarr, zer)


def _tc_combine(agg, x, W_in, b_in, W_out, b_out, W_root, b_root):
    blk = 400
    grid = (N_NODES // blk,)

    def body(agg_ref, x_ref, wi_ref, wo_ref, wr_ref, b_ref, o_ref):
        dot = functools.partial(
            lax.dot_general,
            dimension_numbers=(((1,), (0,)), ((), ())),
            precision=lax.Precision.HIGHEST,
            preferred_element_type=jnp.float32,
        )
        acc = dot(x_ref[...], wr_ref[...])
        for d, w_ref in ((0, wi_ref), (1, wo_ref)):
            cnt = agg_ref[d, 0, :, HALF:HALF + 1]
            scale = 0.5 / jnp.maximum(cnt, 1.0)
            for c in range(2):
                h = agg_ref[d, c, :, 0:HALF] * scale
                acc = acc + dot(h, w_ref[c * HALF:(c + 1) * HALF, :])
        o_ref[...] = acc + b_ref[...]

    bias = (0.5 * b_in + 0.5 * b_out + b_root).reshape(1, D)
    return pl.pallas_call(
        body,
        grid=grid,
        in_specs=[
            pl.BlockSpec((2, NC, blk, ROWW), lambda i: (0, 0, i, 0)),

            pl.BlockSpec((blk, D), lambda i: (i, 0)),
            pl.BlockSpec((D, D), lambda i: (0, 0)),
            pl.BlockSpec((D, D), lambda i: (0, 0)),
            pl.BlockSpec((D, D), lambda i: (0, 0)),
            pl.BlockSpec((1, D), lambda i: (0, 0)),
        ],
        out_specs=pl.BlockSpec((blk, D), lambda i: (i, 0)),
        out_shape=jax.ShapeDtypeStruct((N_NODES, D), jnp.float32),
    )(agg, x, W_in, W_out, W_root, bias)


def kernel(x, edge_index, W_in, b_in, W_out, b_out, W_root, b_root):
    n_edges = edge_index.shape[1]
    src = edge_index[0].astype(jnp.int32)
    dst = edge_index[1].astype(jnp.int32)

    ones = jnp.ones((N_NODES, 1), jnp.float32)
    zpad = jnp.zeros((N_NODES, ROWW - HALF - 1), jnp.float32)
    xcat = jnp.concatenate([
        jnp.concatenate([x[:, :HALF], ones, zpad], axis=1),
        jnp.concatenate([x[:, HALF:], ones, zpad], axis=1),
        jnp.zeros((8, ROWW), jnp.float32),
    ], axis=0)

    ept = ((n_edges // NS) + 2 * B - 1) // (2 * B) * (2 * B)
    n_batches = ept // B
    e_pad = NS * ept
    pad = e_pad - n_edges
    pad_g = 2 * N_NODES + (jnp.arange(pad, dtype=jnp.int32) % 8)
    pad_t = jnp.arange(pad, dtype=jnp.int32) % N_NODES

    def tiled(idx, pad_vals):
        return jnp.concatenate([idx, pad_vals]).reshape(NS, n_batches, B)

    garr = jnp.stack([
        jnp.stack([tiled(src, pad_g), tiled(src + N_NODES, pad_g)]),
        jnp.stack([tiled(dst, pad_g), tiled(dst + N_NODES, pad_g)]),
    ])
    tarr = jnp.stack([tiled(dst, pad_t), tiled(src, pad_t)])

    zer = jnp.zeros((CHUNK, ROWW), jnp.float32)
    agg = _sc_segment_sums(xcat, garr, tarr, zer, n_batches)
    return _tc_combine(agg, x, W_in, b_in, W_out, b_out, W_root, b_root)
